# baseline (speedup 1.0000x reference)
import functools

import jax
import jax.numpy as jnp
from jax import lax
from jax.experimental import pallas as pl
from jax.experimental.pallas import tpu as pltpu
import jax.experimental.pallas.tpu_sc as plsc

N_PTS = 500000
N_VOX = 100000
N_BOX = 2048
C = 64

NWORK = 32
VPW = 400
NWIN = N_VOX // VPW
BPW = N_BOX // NWORK
CH = 512
RBUF = CH + 8
IDS_BUF = 520
UNR = 8
W32 = C // 2

_MESH = plsc.VectorSubcoreMesh(
    core_axis_name="c", subcore_axis_name="s", num_cores=2, num_subcores=16)


BLK1 = 10000


def _pt_mlp_body(coords, feats, w1, b1, w2, b2, pw, pb, fw, fb, out):
    pe1 = jnp.maximum(coords[...] @ w1[...] + b1[...], 0.0)
    pe = pe1 @ w2[...] + b2[...]
    fe = jnp.maximum(feats[...] @ pw[...] + pb[...], 0.0)
    fwm = fw[...]
    h = fe @ fwm[0:32, :] + pe @ fwm[32:64, :] + fb[...]
    h = jnp.maximum(h, 0.0)
    u = lax.bitcast_convert_type(h, jnp.uint32)
    ulo = u[:, 0:W32]
    uhi = u[:, W32:C]
    rlo = ulo + jnp.uint32(0x7FFF) + ((ulo >> jnp.uint32(16)) & jnp.uint32(1))
    rhi = uhi + jnp.uint32(0x7FFF) + ((uhi >> jnp.uint32(16)) & jnp.uint32(1))
    out[...] = (rlo >> jnp.uint32(16)) | (rhi & jnp.uint32(0xFFFF0000))


def _pt_mlp(norm_coords, features, w1, b1, w2, b2, pw, pb, fw, fb):
    grid = N_PTS // BLK1
    full = lambda shape: pl.BlockSpec(shape, lambda i: (0, 0))
    return pl.pallas_call(
        _pt_mlp_body,
        grid=(grid,),
        in_specs=[
            pl.BlockSpec((BLK1, 3), lambda i: (i, 0)),
            pl.BlockSpec((BLK1, C), lambda i: (i, 0)),
            full((3, 32)), full((1, 32)), full((32, 32)), full((1, 32)),
            full((C, 32)), full((1, 32)), full((C, C)), full((1, C)),
        ],
        out_specs=pl.BlockSpec((BLK1, W32), lambda i: (i, 0)),
        out_shape=jax.ShapeDtypeStruct((N_PTS, W32), jnp.uint32),
    )(norm_coords, features, w1, b1, w2, b2, pw, pb, fw, fb)


@functools.partial(
    pl.kernel,
    out_type=jax.ShapeDtypeStruct((N_VOX, W32), jnp.uint32),
    mesh=_MESH,
    scratch_types=[
        pltpu.VMEM((NWIN + 22,), jnp.int32),
        pltpu.VMEM((RBUF, W32), jnp.uint32),
        pltpu.VMEM((IDS_BUF + 16,), jnp.int32),
        pltpu.VMEM((VPW, W32), jnp.uint32),
    ],
)
def _segmax_sc(embs_hbm, ids_hbm, wstart_hbm, zeros_hbm, out_hbm,
               ws_v, rows_v, ids_v, acc_v):
    wid = lax.axis_index("s") * 2 + lax.axis_index("c")
    pltpu.sync_copy(wstart_hbm, ws_v)
    wlo = (wid * NWIN) // NWORK
    whi = ((wid + 1) * NWIN) // NWORK

    zvec = jnp.zeros((16,), jnp.uint32)

    def win_body(win, _):
        ab = ws_v[pl.ds(win, 16)]
        a = ab[0]
        b = ab[1]
        base = win * VPW
        pltpu.sync_copy(zeros_hbm, acc_v)
        nch = (b - a + CH - 1) // CH

        def step(vid, prev, run, ro):
            keep = jnp.where(vid == prev, jnp.uint32(0xFFFFFFFF),
                             jnp.uint32(0))
            mh = jnp.uint32(0xFFFF0000)
            ml = jnp.uint32(0x0000FFFF)
            new_run = []
            for q in range(W32 // 16):
                a = run[q] & keep
                b = rows_v[ro, pl.ds(q * 16, 16)]
                nr = (jnp.maximum(a & mh, b & mh)
                      | jnp.maximum(a & ml, b & ml))
                acc_v[vid - base, pl.ds(q * 16, 16)] = nr
                new_run.append(nr)
            return vid, tuple(new_run)

        def chunk_body(ci, carry, a=a, b=b, base=base):
            prev, run = carry
            s0 = a + ci * CH
            e = jnp.minimum(s0 + CH, b)
            lb = jnp.minimum((s0 // 8) * 8, N_PTS - RBUF)
            pltpu.sync_copy(embs_hbm.at[pl.ds(lb, RBUF), :], rows_v)
            pltpu.sync_copy(ids_hbm.at[pl.ds(lb, IDS_BUF)],
                            ids_v.at[pl.ds(0, IDS_BUF)])
            roff = s0 - lb
            cnt = e - s0
            ngrp = cnt // UNR

            def grp_body(g, carry):
                prev, run = carry
                j0 = roff + g * UNR
                idv = ids_v[pl.ds(j0, 16)]
                for u in range(UNR):
                    prev, run = step(idv[u], prev, run, j0 + u)
                return prev, run

            prev, run = lax.fori_loop(0, ngrp, grp_body, (prev, run))

            def pt_body(j, carry):
                prev, run = carry
                ro = roff + j
                return step(ids_v[pl.ds(ro, 16)][0], prev, run, ro)

            return lax.fori_loop(ngrp * UNR, cnt, pt_body, (prev, run))

        lax.fori_loop(0, nch, chunk_body, (-1, (zvec,) * (W32 // 16)))
        pltpu.sync_copy(acc_v, out_hbm.at[pl.ds(base, VPW)])
        return _

    lax.fori_loop(wlo, whi, win_body, None)


BLK3 = 5000


def _vox_body(vf, vp, table, aw1, ab1, aw2, ab2, out):
    p = vp[...]
    flat = p[:, 0] * 36 + p[:, 1] * 6 + p[:, 2]
    iot = lax.broadcasted_iota(jnp.int32, (BLK3, 216), 1)
    onehot = jnp.where(iot == flat[:, None], 1.0, 0.0)
    pe = onehot @ table[...]
    pk = vf[...]
    lo = lax.bitcast_convert_type(pk << jnp.uint32(16), jnp.float32)
    hi = lax.bitcast_convert_type(pk & jnp.uint32(0xFFFF0000), jnp.float32)
    ve = jnp.concatenate([lo, hi], axis=1) + pe
    h = jnp.maximum(ve @ aw1[...] + ab1[...], 0.0)
    wgt = jax.nn.sigmoid(jnp.sum(h * aw2[...], axis=1, keepdims=True)
                         + ab2[...])
    out[...] = wgt * ve


def _vox_stage(vox_feat, vox_pos, table, aw1, ab1, aw2, ab2):
    grid = N_VOX // BLK3
    full = lambda shape: pl.BlockSpec(shape, lambda i: (0, 0))
    return pl.pallas_call(
        _vox_body,
        grid=(grid,),
        in_specs=[
            pl.BlockSpec((BLK3, W32), lambda i: (i, 0)),
            pl.BlockSpec((BLK3, 3), lambda i: (i, 0)),
            full((216, C)), full((C, 32)), full((1, 32)),
            full((1, 32)), full((1, 1)),
        ],
        out_specs=pl.BlockSpec((BLK3, C), lambda i: (i, 0)),
        out_shape=jax.ShapeDtypeStruct((N_VOX, C), jnp.float32),
    )(vox_feat, vox_pos, table, aw1, ab1, aw2, ab2)


@functools.partial(
    pl.kernel,
    out_type=jax.ShapeDtypeStruct((N_BOX, C), jnp.float32),
    mesh=_MESH,
    scratch_types=[
        pltpu.VMEM((48,), jnp.int32),
        pltpu.VMEM((RBUF, C), jnp.float32),
        pltpu.VMEM((IDS_BUF + 16,), jnp.int32),
        pltpu.VMEM((BPW, C), jnp.float32),
    ],
)
def _segsum_sc(wh_hbm, ids_hbm, bstart_hbm, zeros_hbm, out_hbm,
               bs_v, rows_v, ids_v, acc_v):
    wid = lax.axis_index("s") * 2 + lax.axis_index("c")
    pltpu.sync_copy(bstart_hbm, bs_v)
    ab = bs_v[pl.ds(wid, 16)]
    a = ab[0]
    b = ab[1]
    base = wid * BPW
    pltpu.sync_copy(zeros_hbm, acc_v)
    nch = (b - a + CH - 1) // CH
    zvec = jnp.zeros((16,), jnp.float32)

    def step(vid, prev, run, ro):
        keep = jnp.where(vid == prev, 1.0, 0.0)
        new_run = tuple(
            run[q] * keep + rows_v[ro, pl.ds(q * 16, 16)]
            for q in range(C // 16))
        for q in range(C // 16):
            acc_v[vid - base, pl.ds(q * 16, 16)] = new_run[q]
        return vid, new_run

    def chunk_body(ci, carry):
        prev, run = carry
        s0 = a + ci * CH
        e = jnp.minimum(s0 + CH, b)
        lb = jnp.minimum((s0 // 8) * 8, N_VOX - RBUF)
        pltpu.sync_copy(wh_hbm.at[pl.ds(lb, RBUF), :], rows_v)
        pltpu.sync_copy(ids_hbm.at[pl.ds(lb, IDS_BUF)],
                        ids_v.at[pl.ds(0, IDS_BUF)])
        roff = s0 - lb
        cnt = e - s0
        ngrp = cnt // UNR

        def grp_body(g, carry):
            prev, run = carry
            j0 = roff + g * UNR
            idv = ids_v[pl.ds(j0, 16)]
            for u in range(UNR):
                prev, run = step(idv[u], prev, run, j0 + u)
            return prev, run

        prev, run = lax.fori_loop(0, ngrp, grp_body, (prev, run))

        def pt_body(j, carry):
            prev, run = carry
            ro = roff + j
            return step(ids_v[pl.ds(ro, 16)][0], prev, run, ro)

        return lax.fori_loop(ngrp * UNR, cnt, pt_body, (prev, run))

    lax.fori_loop(0, nch, chunk_body, (-1, (zvec,) * (C // 16)))
    pltpu.sync_copy(acc_v, out_hbm.at[pl.ds(base, BPW)])


def _head_body(agg, ow, ob, iw, rw, out):
    o = jnp.maximum(agg[...] @ ow[...] + ob[...], 0.0)
    out[...] = jnp.concatenate([o @ iw[...], o @ rw[...]], axis=1)


def _head(agg, ow, ob, iw, rw):
    return pl.pallas_call(
        _head_body,
        out_shape=jax.ShapeDtypeStruct((N_BOX, 9), jnp.float32),
    )(agg, ow, ob, iw, rw)


def kernel(features, norm_coords, pt2vox, vox_pos, vox2box, num_box,
           grid_emb, pos_W1, pos_b1, pos_W2, pos_b2, proj_W, proj_b,
           fc_W, fc_b, attn_W1, attn_b1, attn_W2, attn_b2,
           out_W, out_b, iou_W, reg_W):
    pt2vox = pt2vox.astype(jnp.int32)
    box_ids = jnp.minimum(vox2box, num_box - 1).astype(jnp.int32)

    pt_embs = _pt_mlp(
        norm_coords, features,
        pos_W1, pos_b1.reshape(1, 32), pos_W2, pos_b2.reshape(1, 32),
        proj_W, proj_b.reshape(1, 32), fc_W, fc_b.reshape(1, C))

    wbounds = jnp.searchsorted(
        pt2vox, jnp.arange(NWIN + 1, dtype=jnp.int32) * VPW).astype(jnp.int32)
    wstart = jnp.concatenate([wbounds, jnp.zeros((21,), jnp.int32)])
    zeros = jnp.zeros((VPW, W32), jnp.uint32)

    vox_feat = _segmax_sc(pt_embs, pt2vox, wstart, zeros)

    weighted = _vox_stage(
        vox_feat, vox_pos, grid_emb.reshape(216, C),
        attn_W1, attn_b1.reshape(1, 32), attn_W2.reshape(1, 32),
        attn_b2.reshape(1, 1))

    bbounds = jnp.searchsorted(
        box_ids, jnp.arange(NWORK + 1, dtype=jnp.int32) * BPW).astype(jnp.int32)
    bstart = jnp.concatenate([bbounds, jnp.zeros((15,), jnp.int32)])

    agg = _segsum_sc(weighted, box_ids, bstart,
                     jnp.zeros((BPW, C), jnp.float32))

    return _head(agg, out_W, out_b.reshape(1, 32), iou_W, reg_W)


# speedup vs baseline: 1.2245x; 1.2245x over previous
import functools

import jax
import jax.numpy as jnp
from jax import lax
from jax.experimental import pallas as pl
from jax.experimental.pallas import tpu as pltpu
import jax.experimental.pallas.tpu_sc as plsc

N_PTS = 500000
N_VOX = 100000
N_BOX = 2048
C = 64

NWORK = 32
VPW = 400
NWIN = N_VOX // VPW
BPW = N_BOX // NWORK
CH = 256
RBUF = CH + 8
IDS_BUF = 264
UNR = 8

_MESH = plsc.VectorSubcoreMesh(
    core_axis_name="c", subcore_axis_name="s", num_cores=2, num_subcores=16)


BLK1 = 10000


def _pt_mlp_body(coords, feats, w1, b1, w2, b2, pw, pb, fw, fb, out):
    pe1 = jnp.maximum(coords[...] @ w1[...] + b1[...], 0.0)
    pe = pe1 @ w2[...] + b2[...]
    fe = jnp.maximum(feats[...] @ pw[...] + pb[...], 0.0)
    fwm = fw[...]
    h = fe @ fwm[0:32, :] + pe @ fwm[32:64, :] + fb[...]
    out[...] = jnp.maximum(h, 0.0)


def _pt_mlp(norm_coords, features, w1, b1, w2, b2, pw, pb, fw, fb):
    grid = N_PTS // BLK1
    full = lambda shape: pl.BlockSpec(shape, lambda i: (0, 0))
    return pl.pallas_call(
        _pt_mlp_body,
        grid=(grid,),
        in_specs=[
            pl.BlockSpec((BLK1, 3), lambda i: (i, 0)),
            pl.BlockSpec((BLK1, C), lambda i: (i, 0)),
            full((3, 32)), full((1, 32)), full((32, 32)), full((1, 32)),
            full((C, 32)), full((1, 32)), full((C, C)), full((1, C)),
        ],
        out_specs=pl.BlockSpec((BLK1, C), lambda i: (i, 0)),
        out_shape=jax.ShapeDtypeStruct((N_PTS, C), jnp.float32),
    )(norm_coords, features, w1, b1, w2, b2, pw, pb, fw, fb)


@functools.partial(
    pl.kernel,
    out_type=jax.ShapeDtypeStruct((N_VOX, C), jnp.float32),
    mesh=_MESH,
    scratch_types=[
        pltpu.VMEM((NWIN + 22,), jnp.int32),
        pltpu.VMEM((2 * RBUF, C), jnp.float32),
        pltpu.VMEM((2 * (IDS_BUF + 16),), jnp.int32),
        pltpu.VMEM((VPW, C), jnp.float32),
        pltpu.SemaphoreType.DMA((2,)),
        pltpu.SemaphoreType.DMA((2,)),
    ],
)
def _segmax_sc(embs_hbm, ids_hbm, wstart_hbm, zeros_hbm, out_hbm,
               ws_v, rows_v, ids_v, acc_v, semr, semi):
    wid = lax.axis_index("s") * 2 + lax.axis_index("c")
    pltpu.sync_copy(wstart_hbm, ws_v)
    wlo = (wid * NWIN) // NWORK
    whi = ((wid + 1) * NWIN) // NWORK

    zvec = jnp.zeros((16,), jnp.float32)

    def win_body(win, _):
        ab = ws_v[pl.ds(win, 16)]
        a = ab[0]
        b = ab[1]
        base = win * VPW
        pltpu.sync_copy(zeros_hbm, acc_v)
        nch = (b - a + CH - 1) // CH

        def start_chunk(ci, a=a):
            s0 = a + ci * CH
            lb = jnp.minimum((s0 // 8) * 8, N_PTS - RBUF)
            par = ci & 1
            pltpu.async_copy(embs_hbm.at[pl.ds(lb, RBUF), :],
                             rows_v.at[pl.ds(par * RBUF, RBUF), :],
                             semr.at[par])
            pltpu.async_copy(ids_hbm.at[pl.ds(lb, IDS_BUF)],
                             ids_v.at[pl.ds(par * (IDS_BUF + 16), IDS_BUF)],
                             semi.at[par])

        @pl.when(nch > 0)
        def _():
            start_chunk(0)

        def step(vid, prev, run, ro, rbase):
            keep = jnp.where(vid == prev, 1.0, 0.0)
            new_run = tuple(
                jnp.maximum(run[q] * keep,
                            rows_v[rbase + ro, pl.ds(q * 16, 16)])
                for q in range(C // 16))
            for q in range(C // 16):
                acc_v[vid - base, pl.ds(q * 16, 16)] = new_run[q]
            return vid, new_run

        def chunk_body(ci, carry, a=a, b=b, base=base):
            prev, run = carry
            par = ci & 1
            rbase = par * RBUF
            ibase = par * (IDS_BUF + 16)
            s0 = a + ci * CH
            e = jnp.minimum(s0 + CH, b)
            lb = jnp.minimum((s0 // 8) * 8, N_PTS - RBUF)
            pltpu.make_async_copy(embs_hbm.at[pl.ds(lb, RBUF), :],
                                  rows_v.at[pl.ds(rbase, RBUF), :],
                                  semr.at[par]).wait()
            pltpu.make_async_copy(ids_hbm.at[pl.ds(lb, IDS_BUF)],
                                  ids_v.at[pl.ds(ibase, IDS_BUF)],
                                  semi.at[par]).wait()

            @pl.when(ci + 1 < nch)
            def _():
                start_chunk(ci + 1)

            roff = s0 - lb
            cnt = e - s0
            ngrp = cnt // UNR

            def grp_body(g, carry):
                prev, run = carry
                j0 = roff + g * UNR
                idv = ids_v[pl.ds(ibase + j0, 16)]
                for u in range(UNR):
                    prev, run = step(idv[u], prev, run, j0 + u, rbase)
                return prev, run

            prev, run = lax.fori_loop(0, ngrp, grp_body, (prev, run))

            def pt_body(j, carry):
                prev, run = carry
                ro = roff + j
                return step(ids_v[pl.ds(ibase + ro, 16)][0], prev, run, ro,
                            rbase)

            return lax.fori_loop(ngrp * UNR, cnt, pt_body, (prev, run))

        lax.fori_loop(0, nch, chunk_body, (-1, (zvec,) * (C // 16)))
        pltpu.sync_copy(acc_v, out_hbm.at[pl.ds(base, VPW)])
        return _

    lax.fori_loop(wlo, whi, win_body, None)


BLK3 = 5000


def _vox_body(vf, vp, table, aw1, ab1, aw2, ab2, out):
    p = vp[...]
    flat = p[:, 0] * 36 + p[:, 1] * 6 + p[:, 2]
    iot = lax.broadcasted_iota(jnp.int32, (BLK3, 216), 1)
    onehot = jnp.where(iot == flat[:, None], 1.0, 0.0)
    pe = onehot @ table[...]
    ve = vf[...] + pe
    h = jnp.maximum(ve @ aw1[...] + ab1[...], 0.0)
    wgt = jax.nn.sigmoid(jnp.sum(h * aw2[...], axis=1, keepdims=True)
                         + ab2[...])
    out[...] = wgt * ve


def _vox_stage(vox_feat, vox_pos, table, aw1, ab1, aw2, ab2):
    grid = N_VOX // BLK3
    full = lambda shape: pl.BlockSpec(shape, lambda i: (0, 0))
    return pl.pallas_call(
        _vox_body,
        grid=(grid,),
        in_specs=[
            pl.BlockSpec((BLK3, C), lambda i: (i, 0)),
            pl.BlockSpec((BLK3, 3), lambda i: (i, 0)),
            full((216, C)), full((C, 32)), full((1, 32)),
            full((1, 32)), full((1, 1)),
        ],
        out_specs=pl.BlockSpec((BLK3, C), lambda i: (i, 0)),
        out_shape=jax.ShapeDtypeStruct((N_VOX, C), jnp.float32),
    )(vox_feat, vox_pos, table, aw1, ab1, aw2, ab2)


@functools.partial(
    pl.kernel,
    out_type=jax.ShapeDtypeStruct((N_BOX, C), jnp.float32),
    mesh=_MESH,
    scratch_types=[
        pltpu.VMEM((48,), jnp.int32),
        pltpu.VMEM((RBUF, C), jnp.float32),
        pltpu.VMEM((IDS_BUF + 16,), jnp.int32),
        pltpu.VMEM((BPW, C), jnp.float32),
    ],
)
def _segsum_sc(wh_hbm, ids_hbm, bstart_hbm, zeros_hbm, out_hbm,
               bs_v, rows_v, ids_v, acc_v):
    wid = lax.axis_index("s") * 2 + lax.axis_index("c")
    pltpu.sync_copy(bstart_hbm, bs_v)
    ab = bs_v[pl.ds(wid, 16)]
    a = ab[0]
    b = ab[1]
    base = wid * BPW
    pltpu.sync_copy(zeros_hbm.at[pl.ds(0, BPW), :], acc_v)
    nch = (b - a + CH - 1) // CH
    zvec = jnp.zeros((16,), jnp.float32)

    def step(vid, prev, run, ro):
        keep = jnp.where(vid == prev, 1.0, 0.0)
        new_run = tuple(
            run[q] * keep + rows_v[ro, pl.ds(q * 16, 16)]
            for q in range(C // 16))
        for q in range(C // 16):
            acc_v[vid - base, pl.ds(q * 16, 16)] = new_run[q]
        return vid, new_run

    def chunk_body(ci, carry):
        prev, run = carry
        s0 = a + ci * CH
        e = jnp.minimum(s0 + CH, b)
        lb = jnp.minimum((s0 // 8) * 8, N_VOX - RBUF)
        pltpu.sync_copy(wh_hbm.at[pl.ds(lb, RBUF), :], rows_v)
        pltpu.sync_copy(ids_hbm.at[pl.ds(lb, IDS_BUF)],
                        ids_v.at[pl.ds(0, IDS_BUF)])
        roff = s0 - lb
        cnt = e - s0
        ngrp = cnt // UNR

        def grp_body(g, carry):
            prev, run = carry
            j0 = roff + g * UNR
            idv = ids_v[pl.ds(j0, 16)]
            for u in range(UNR):
                prev, run = step(idv[u], prev, run, j0 + u)
            return prev, run

        prev, run = lax.fori_loop(0, ngrp, grp_body, (prev, run))

        def pt_body(j, carry):
            prev, run = carry
            ro = roff + j
            return step(ids_v[pl.ds(ro, 16)][0], prev, run, ro)

        return lax.fori_loop(ngrp * UNR, cnt, pt_body, (prev, run))

    lax.fori_loop(0, nch, chunk_body, (-1, (zvec,) * (C // 16)))
    pltpu.sync_copy(acc_v, out_hbm.at[pl.ds(base, BPW)])


def _head_body(agg, ow, ob, iw, rw, out):
    o = jnp.maximum(agg[...] @ ow[...] + ob[...], 0.0)
    out[...] = jnp.concatenate([o @ iw[...], o @ rw[...]], axis=1)


def _head(agg, ow, ob, iw, rw):
    return pl.pallas_call(
        _head_body,
        out_shape=jax.ShapeDtypeStruct((N_BOX, 9), jnp.float32),
    )(agg, ow, ob, iw, rw)


def kernel(features, norm_coords, pt2vox, vox_pos, vox2box, num_box,
           grid_emb, pos_W1, pos_b1, pos_W2, pos_b2, proj_W, proj_b,
           fc_W, fc_b, attn_W1, attn_b1, attn_W2, attn_b2,
           out_W, out_b, iou_W, reg_W):
    pt2vox = pt2vox.astype(jnp.int32)
    box_ids = jnp.minimum(vox2box, num_box - 1).astype(jnp.int32)

    pt_embs = _pt_mlp(
        norm_coords, features,
        pos_W1, pos_b1.reshape(1, 32), pos_W2, pos_b2.reshape(1, 32),
        proj_W, proj_b.reshape(1, 32), fc_W, fc_b.reshape(1, C))

    wbounds = jnp.searchsorted(
        pt2vox, jnp.arange(NWIN + 1, dtype=jnp.int32) * VPW).astype(jnp.int32)
    wstart = jnp.concatenate([wbounds, jnp.zeros((21,), jnp.int32)])
    zeros = jnp.zeros((VPW, C), jnp.float32)

    vox_feat = _segmax_sc(pt_embs, pt2vox, wstart, zeros)

    weighted = _vox_stage(
        vox_feat, vox_pos, grid_emb.reshape(216, C),
        attn_W1, attn_b1.reshape(1, 32), attn_W2.reshape(1, 32),
        attn_b2.reshape(1, 1))

    bbounds = jnp.searchsorted(
        box_ids, jnp.arange(NWORK + 1, dtype=jnp.int32) * BPW).astype(jnp.int32)
    bstart = jnp.concatenate([bbounds, jnp.zeros((15,), jnp.int32)])

    agg = _segsum_sc(weighted, box_ids, bstart, zeros)

    return _head(agg, out_W, out_b.reshape(1, 32), iou_W, reg_W)


# speedup vs baseline: 1.2626x; 1.0311x over previous
import functools

import jax
import jax.numpy as jnp
from jax import lax
from jax.experimental import pallas as pl
from jax.experimental.pallas import tpu as pltpu
import jax.experimental.pallas.tpu_sc as plsc

N_PTS = 500000
N_VOX = 100000
N_BOX = 2048
C = 64

NWORK = 32
VPW = 400
NWIN = N_VOX // VPW
BPW = N_BOX // NWORK
CH = 256
RBUF = CH + 8
IDS_BUF = 264
UNR = 8

_MESH = plsc.VectorSubcoreMesh(
    core_axis_name="c", subcore_axis_name="s", num_cores=2, num_subcores=16)


BLK1 = 10000


def _pt_mlp_body(coords, feats, w1, b1, w2, b2, pw, pb, fw, fb, out):
    pe1 = jnp.maximum(coords[...] @ w1[...] + b1[...], 0.0)
    pe = pe1 @ w2[...] + b2[...]
    fe = jnp.maximum(feats[...] @ pw[...] + pb[...], 0.0)
    fwm = fw[...]
    h = fe @ fwm[0:32, :] + pe @ fwm[32:64, :] + fb[...]
    out[...] = jnp.maximum(h, 0.0)


def _pt_mlp(norm_coords, features, w1, b1, w2, b2, pw, pb, fw, fb):
    grid = N_PTS // BLK1
    full = lambda shape: pl.BlockSpec(shape, lambda i: (0, 0))
    return pl.pallas_call(
        _pt_mlp_body,
        grid=(grid,),
        in_specs=[
            pl.BlockSpec((BLK1, 3), lambda i: (i, 0)),
            pl.BlockSpec((BLK1, C), lambda i: (i, 0)),
            full((3, 32)), full((1, 32)), full((32, 32)), full((1, 32)),
            full((C, 32)), full((1, 32)), full((C, C)), full((1, C)),
        ],
        out_specs=pl.BlockSpec((BLK1, C), lambda i: (i, 0)),
        out_shape=jax.ShapeDtypeStruct((N_PTS, C), jnp.float32),
    )(norm_coords, features, w1, b1, w2, b2, pw, pb, fw, fb)


@functools.partial(
    pl.kernel,
    out_type=jax.ShapeDtypeStruct((N_VOX, C), jnp.float32),
    mesh=_MESH,
    scratch_types=[
        pltpu.VMEM((NWIN + 22,), jnp.int32),
        pltpu.VMEM((2 * RBUF, C), jnp.float32),
        pltpu.VMEM((2 * (IDS_BUF + 16),), jnp.int32),
        pltpu.VMEM((VPW, C), jnp.float32),
        pltpu.SemaphoreType.DMA((2,)),
        pltpu.SemaphoreType.DMA((2,)),
    ],
)
def _segmax_sc(embs_hbm, ids_hbm, wstart_hbm, zeros_hbm, out_hbm,
               ws_v, rows_v, ids_v, acc_v, semr, semi):
    wid = lax.axis_index("s") * 2 + lax.axis_index("c")
    pltpu.sync_copy(wstart_hbm, ws_v)
    wlo = (wid * NWIN) // NWORK
    whi = ((wid + 1) * NWIN) // NWORK

    zvec = jnp.zeros((16,), jnp.float32)

    def win_body(win, _):
        ab = ws_v[pl.ds(win, 16)]
        a = ab[0]
        b = ab[1]
        base = win * VPW
        pltpu.sync_copy(zeros_hbm, acc_v)
        nch = (b - a + CH - 1) // CH

        def start_chunk(ci, a=a):
            s0 = a + ci * CH
            lb = jnp.minimum((s0 // 8) * 8, N_PTS - RBUF)
            par = ci & 1
            pltpu.async_copy(embs_hbm.at[pl.ds(lb, RBUF), :],
                             rows_v.at[pl.ds(par * RBUF, RBUF), :],
                             semr.at[par])
            pltpu.async_copy(ids_hbm.at[pl.ds(lb, IDS_BUF)],
                             ids_v.at[pl.ds(par * (IDS_BUF + 16), IDS_BUF)],
                             semi.at[par])

        @pl.when(nch > 0)
        def _():
            start_chunk(0)

        def step(vid, prev, run, ro, rbase):
            keep = jnp.where(vid == prev, 1.0, 0.0)
            new_run = tuple(
                jnp.maximum(run[q] * keep,
                            rows_v[rbase + ro, pl.ds(q * 16, 16)])
                for q in range(C // 16))
            for q in range(C // 16):
                acc_v[vid - base, pl.ds(q * 16, 16)] = new_run[q]
            return vid, new_run

        def chunk_body(ci, carry, a=a, b=b, base=base):
            prev, run = carry
            par = ci & 1
            rbase = par * RBUF
            ibase = par * (IDS_BUF + 16)
            s0 = a + ci * CH
            e = jnp.minimum(s0 + CH, b)
            lb = jnp.minimum((s0 // 8) * 8, N_PTS - RBUF)
            pltpu.make_async_copy(embs_hbm.at[pl.ds(lb, RBUF), :],
                                  rows_v.at[pl.ds(rbase, RBUF), :],
                                  semr.at[par]).wait()
            pltpu.make_async_copy(ids_hbm.at[pl.ds(lb, IDS_BUF)],
                                  ids_v.at[pl.ds(ibase, IDS_BUF)],
                                  semi.at[par]).wait()

            @pl.when(ci + 1 < nch)
            def _():
                start_chunk(ci + 1)

            roff = s0 - lb
            cnt = e - s0
            ngrp = cnt // UNR

            def grp_body(g, carry):
                prev, run = carry
                j0 = roff + g * UNR
                idv = ids_v[pl.ds(ibase + j0, 16)]
                for u in range(UNR):
                    prev, run = step(idv[u], prev, run, j0 + u, rbase)
                return prev, run

            prev, run = lax.fori_loop(0, ngrp, grp_body, (prev, run))

            def pt_body(j, carry):
                prev, run = carry
                ro = roff + j
                return step(ids_v[pl.ds(ibase + ro, 16)][0], prev, run, ro,
                            rbase)

            return lax.fori_loop(ngrp * UNR, cnt, pt_body, (prev, run))

        lax.fori_loop(0, nch, chunk_body, (-1, (zvec,) * (C // 16)))
        pltpu.sync_copy(acc_v, out_hbm.at[pl.ds(base, VPW)])
        return _

    lax.fori_loop(wlo, whi, win_body, None)


BLK3 = 5000


def _vox_body(vf, vp, table, aw1, ab1, aw2, ab2, out):
    p = vp[...]
    flat = p[:, 0] * 36 + p[:, 1] * 6 + p[:, 2]
    iot = lax.broadcasted_iota(jnp.int32, (BLK3, 216), 1)
    onehot = jnp.where(iot == flat[:, None], 1.0, 0.0)
    pe = onehot @ table[...]
    ve = vf[...] + pe
    h = jnp.maximum(ve @ aw1[...] + ab1[...], 0.0)
    wgt = jax.nn.sigmoid(jnp.sum(h * aw2[...], axis=1, keepdims=True)
                         + ab2[...])
    out[...] = wgt * ve


def _vox_stage(vox_feat, vox_pos, table, aw1, ab1, aw2, ab2):
    grid = N_VOX // BLK3
    full = lambda shape: pl.BlockSpec(shape, lambda i: (0, 0))
    return pl.pallas_call(
        _vox_body,
        grid=(grid,),
        in_specs=[
            pl.BlockSpec((BLK3, C), lambda i: (i, 0)),
            pl.BlockSpec((BLK3, 3), lambda i: (i, 0)),
            full((216, C)), full((C, 32)), full((1, 32)),
            full((1, 32)), full((1, 1)),
        ],
        out_specs=pl.BlockSpec((BLK3, C), lambda i: (i, 0)),
        out_shape=jax.ShapeDtypeStruct((N_VOX, C), jnp.float32),
    )(vox_feat, vox_pos, table, aw1, ab1, aw2, ab2)


@functools.partial(
    pl.kernel,
    out_type=jax.ShapeDtypeStruct((N_BOX, C), jnp.float32),
    mesh=_MESH,
    scratch_types=[
        pltpu.VMEM((48,), jnp.int32),
        pltpu.VMEM((2 * RBUF, C), jnp.float32),
        pltpu.VMEM((2 * (IDS_BUF + 16),), jnp.int32),
        pltpu.VMEM((BPW, C), jnp.float32),
        pltpu.SemaphoreType.DMA((2,)),
        pltpu.SemaphoreType.DMA((2,)),
    ],
)
def _segsum_sc(wh_hbm, ids_hbm, bstart_hbm, zeros_hbm, out_hbm,
               bs_v, rows_v, ids_v, acc_v, semr, semi):
    wid = lax.axis_index("s") * 2 + lax.axis_index("c")
    pltpu.sync_copy(bstart_hbm, bs_v)
    ab = bs_v[pl.ds(wid, 16)]
    a = ab[0]
    b = ab[1]
    base = wid * BPW
    pltpu.sync_copy(zeros_hbm.at[pl.ds(0, BPW), :], acc_v)
    nch = (b - a + CH - 1) // CH
    zvec = jnp.zeros((16,), jnp.float32)

    def start_chunk(ci):
        s0 = a + ci * CH
        lb = jnp.minimum((s0 // 8) * 8, N_VOX - RBUF)
        par = ci & 1
        pltpu.async_copy(wh_hbm.at[pl.ds(lb, RBUF), :],
                         rows_v.at[pl.ds(par * RBUF, RBUF), :], semr.at[par])
        pltpu.async_copy(ids_hbm.at[pl.ds(lb, IDS_BUF)],
                         ids_v.at[pl.ds(par * (IDS_BUF + 16), IDS_BUF)],
                         semi.at[par])

    @pl.when(nch > 0)
    def _():
        start_chunk(0)

    def step(vid, prev, run, ro, rbase):
        keep = jnp.where(vid == prev, 1.0, 0.0)
        new_run = tuple(
            run[q] * keep + rows_v[rbase + ro, pl.ds(q * 16, 16)]
            for q in range(C // 16))
        for q in range(C // 16):
            acc_v[vid - base, pl.ds(q * 16, 16)] = new_run[q]
        return vid, new_run

    def chunk_body(ci, carry):
        prev, run = carry
        par = ci & 1
        rbase = par * RBUF
        ibase = par * (IDS_BUF + 16)
        s0 = a + ci * CH
        e = jnp.minimum(s0 + CH, b)
        lb = jnp.minimum((s0 // 8) * 8, N_VOX - RBUF)
        pltpu.make_async_copy(wh_hbm.at[pl.ds(lb, RBUF), :],
                              rows_v.at[pl.ds(rbase, RBUF), :],
                              semr.at[par]).wait()
        pltpu.make_async_copy(ids_hbm.at[pl.ds(lb, IDS_BUF)],
                              ids_v.at[pl.ds(ibase, IDS_BUF)],
                              semi.at[par]).wait()

        @pl.when(ci + 1 < nch)
        def _():
            start_chunk(ci + 1)

        roff = s0 - lb
        cnt = e - s0
        ngrp = cnt // UNR

        def grp_body(g, carry):
            prev, run = carry
            j0 = roff + g * UNR
            idv = ids_v[pl.ds(ibase + j0, 16)]
            for u in range(UNR):
                prev, run = step(idv[u], prev, run, j0 + u, rbase)
            return prev, run

        prev, run = lax.fori_loop(0, ngrp, grp_body, (prev, run))

        def pt_body(j, carry):
            prev, run = carry
            ro = roff + j
            return step(ids_v[pl.ds(ibase + ro, 16)][0], prev, run, ro,
                        rbase)

        return lax.fori_loop(ngrp * UNR, cnt, pt_body, (prev, run))

    lax.fori_loop(0, nch, chunk_body, (-1, (zvec,) * (C // 16)))
    pltpu.sync_copy(acc_v, out_hbm.at[pl.ds(base, BPW)])


def _head_body(agg, ow, ob, iw, rw, out):
    o = jnp.maximum(agg[...] @ ow[...] + ob[...], 0.0)
    out[...] = jnp.concatenate([o @ iw[...], o @ rw[...]], axis=1)


def _head(agg, ow, ob, iw, rw):
    return pl.pallas_call(
        _head_body,
        out_shape=jax.ShapeDtypeStruct((N_BOX, 9), jnp.float32),
    )(agg, ow, ob, iw, rw)


def kernel(features, norm_coords, pt2vox, vox_pos, vox2box, num_box,
           grid_emb, pos_W1, pos_b1, pos_W2, pos_b2, proj_W, proj_b,
           fc_W, fc_b, attn_W1, attn_b1, attn_W2, attn_b2,
           out_W, out_b, iou_W, reg_W):
    pt2vox = pt2vox.astype(jnp.int32)
    box_ids = jnp.minimum(vox2box, num_box - 1).astype(jnp.int32)

    pt_embs = _pt_mlp(
        norm_coords, features,
        pos_W1, pos_b1.reshape(1, 32), pos_W2, pos_b2.reshape(1, 32),
        proj_W, proj_b.reshape(1, 32), fc_W, fc_b.reshape(1, C))

    wbounds = jnp.searchsorted(
        pt2vox, jnp.arange(NWIN + 1, dtype=jnp.int32) * VPW).astype(jnp.int32)
    wstart = jnp.concatenate([wbounds, jnp.zeros((21,), jnp.int32)])
    zeros = jnp.zeros((VPW, C), jnp.float32)

    vox_feat = _segmax_sc(pt_embs, pt2vox, wstart, zeros)

    weighted = _vox_stage(
        vox_feat, vox_pos, grid_emb.reshape(216, C),
        attn_W1, attn_b1.reshape(1, 32), attn_W2.reshape(1, 32),
        attn_b2.reshape(1, 1))

    bbounds = jnp.searchsorted(
        box_ids, jnp.arange(NWORK + 1, dtype=jnp.int32) * BPW).astype(jnp.int32)
    bstart = jnp.concatenate([bbounds, jnp.zeros((15,), jnp.int32)])

    agg = _segsum_sc(weighted, box_ids, bstart, zeros)

    return _head(agg, out_W, out_b.reshape(1, 32), iou_W, reg_W)
